# baseline (speedup 1.0000x reference)
import functools

import jax
import jax.numpy as jnp
from jax import lax
from jax.experimental import pallas as pl
from jax.experimental.pallas import tpu as pltpu
from jax.experimental.pallas import tpu_sc as plsc

N = 10000
NPAD = 10240
E = 320000
D = 128
CHUNK = 128
NROWS = E // CHUNK
WCH = 80
LASTW = 31
LCH = NROWS - LASTW * WCH
GC = 16
NG = WCH // GC
SLAB = NPAD // 16
_SC_MESH = plsc.VectorSubcoreMesh(core_axis_name="c", subcore_axis_name="s")


def _deg_body(ei3, out, dst_v, ones_v, zb_v, deg_sh, sem):
  c = lax.axis_index("c")
  s = lax.axis_index("s")
  w = c * 16 + s

  @pl.when(w < LASTW)
  def _():
    pltpu.sync_copy(ei3.at[1, pl.ds(w * WCH, WCH)], dst_v)

  @pl.when(w == LASTW)
  def _():
    pltpu.sync_copy(ei3.at[1, pl.ds(LASTW * WCH, GC)], dst_v.at[pl.ds(0, GC)])
    pltpu.sync_copy(ei3.at[1, pl.ds(LASTW * WCH + GC, LCH - GC)],
                    dst_v.at[pl.ds(GC, LCH - GC)])

  for t in range(8):
    ones_v[pl.ds(t * 16, 16)] = jnp.ones((16,), jnp.float32)

  @pl.loop(0, SLAB // 16)
  def _zb(i):
    zb_v[pl.ds(i * 16, 16)] = jnp.zeros((16,), jnp.float32)

  pltpu.sync_copy(zb_v, deg_sh.at[pl.ds(s * SLAB, SLAB)])
  plsc.subcore_barrier()

  nch = jnp.where(w == LASTW, LCH, WCH)

  @pl.loop(0, nch)
  def _fire(j):
    pltpu.async_copy(ones_v, deg_sh.at[dst_v.at[j]], sem, add=True)

  @pl.loop(0, nch)
  def _drain(j):
    pltpu.make_async_copy(ones_v, deg_sh.at[dst_v.at[0]], sem).wait()

  plsc.subcore_barrier()
  pltpu.sync_copy(deg_sh.at[pl.ds(s * SLAB, SLAB)],
                  out.at[c, pl.ds(s * SLAB, SLAB)])


_deg_call = functools.partial(
    pl.kernel,
    out_type=jax.ShapeDtypeStruct((2, NPAD), jnp.float32),
    mesh=_SC_MESH,
    scratch_types=[
        pltpu.VMEM((WCH, CHUNK), jnp.int32),
        pltpu.VMEM((CHUNK,), jnp.float32),
        pltpu.VMEM((SLAB,), jnp.float32),
        pltpu.VMEM_SHARED((NPAD,), jnp.float32),
        pltpu.SemaphoreType.DMA,
    ],
)(_deg_body)


def _agg_body(y, ei3, out, src_a, dst_a, src_b, dst_b, buf0, buf1, acc_sh,
              sg0, sg1, ss0, ss1, sia, sib):
  c = lax.axis_index("c")
  s = lax.axis_index("s")
  w = c * 16 + s

  @pl.loop(0, CHUNK)
  def _zrow(r):
    for t in range(8):
      buf0[r, pl.ds(t * 16, 16)] = jnp.zeros((16,), jnp.float32)

  for k in range(SLAB // CHUNK):
    pltpu.sync_copy(buf0, acc_sh.at[pl.ds(s * SLAB + k * CHUNK, CHUNK)])
  plsc.subcore_barrier()

  idx = [(src_a, dst_a, sia), (src_b, dst_b, sib)]

  def refill(g):
    sv, dv, sem = idx[g % 2]
    pltpu.async_copy(ei3.at[0, pl.ds(w * WCH + g * GC, GC)], sv, sem)
    pltpu.async_copy(ei3.at[1, pl.ds(w * WCH + g * GC, GC)], dv, sem)

  def refill_wait(g):
    sv, dv, sem = idx[g % 2]
    pltpu.make_async_copy(ei3.at[0, pl.ds(0, GC)], sv, sem).wait()
    pltpu.make_async_copy(ei3.at[1, pl.ds(0, GC)], dv, sem).wait()

  @pl.when(w < LASTW)
  def _full_worker():
    pltpu.sync_copy(ei3.at[0, pl.ds(w * WCH, GC)], src_a)
    pltpu.sync_copy(ei3.at[1, pl.ds(w * WCH, GC)], dst_a)
    refill(1)
    pltpu.async_copy(y.at[src_a.at[0]], buf0, sg0)
    pltpu.async_copy(y.at[src_a.at[1]], buf1, sg1)

    for g in range(NG):
      sv, dv, _ = idx[g % 2]
      nsv, ndv, _ = idx[(g + 1) % 2]

      @pl.loop(0, GC // 2 - 1)
      def _pipe(i, sv=sv, dv=dv):
        a = 2 * i
        b = a + 1
        pltpu.make_async_copy(y.at[sv.at[0]], buf0, sg0).wait()
        pltpu.async_copy(buf0, acc_sh.at[dv.at[a]], ss0, add=True)
        pltpu.make_async_copy(y.at[sv.at[1]], buf1, sg1).wait()
        pltpu.async_copy(buf1, acc_sh.at[dv.at[b]], ss1, add=True)
        pltpu.make_async_copy(buf0, acc_sh.at[dv.at[0]], ss0).wait()
        pltpu.async_copy(y.at[sv.at[a + 2]], buf0, sg0)
        pltpu.make_async_copy(buf1, acc_sh.at[dv.at[0]], ss1).wait()
        pltpu.async_copy(y.at[sv.at[b + 2]], buf1, sg1)

      pltpu.make_async_copy(y.at[sv.at[0]], buf0, sg0).wait()
      pltpu.async_copy(buf0, acc_sh.at[dv.at[GC - 2]], ss0, add=True)
      pltpu.make_async_copy(y.at[sv.at[1]], buf1, sg1).wait()
      pltpu.async_copy(buf1, acc_sh.at[dv.at[GC - 1]], ss1, add=True)
      pltpu.make_async_copy(buf0, acc_sh.at[dv.at[0]], ss0).wait()
      if g + 1 < NG:
        refill_wait(g + 1)
        pltpu.async_copy(y.at[nsv.at[0]], buf0, sg0)
      pltpu.make_async_copy(buf1, acc_sh.at[dv.at[0]], ss1).wait()
      if g + 1 < NG:
        pltpu.async_copy(y.at[nsv.at[1]], buf1, sg1)
        if g + 2 < NG:
          refill(g + 2)

  @pl.when(w == LASTW)
  def _tail_worker():
    base = LASTW * WCH
    pltpu.sync_copy(ei3.at[0, pl.ds(base, GC)], src_a)
    pltpu.sync_copy(ei3.at[1, pl.ds(base, GC)], dst_a)
    pltpu.sync_copy(ei3.at[0, pl.ds(base + GC, LCH - GC)],
                    src_b.at[pl.ds(0, LCH - GC)])
    pltpu.sync_copy(ei3.at[1, pl.ds(base + GC, LCH - GC)],
                    dst_b.at[pl.ds(0, LCH - GC)])

    @pl.loop(0, GC)
    def _t0(j):
      pltpu.async_copy(y.at[src_a.at[j]], buf0, sg0).wait()
      pltpu.sync_copy(buf0, acc_sh.at[dst_a.at[j]], add=True)

    @pl.loop(0, LCH - GC)
    def _t1(j):
      pltpu.async_copy(y.at[src_b.at[j]], buf0, sg0).wait()
      pltpu.sync_copy(buf0, acc_sh.at[dst_b.at[j]], add=True)

  plsc.subcore_barrier()
  pltpu.sync_copy(acc_sh.at[pl.ds(s * SLAB, SLAB)],
                  out.at[c, pl.ds(s * SLAB, SLAB)])


_agg_call = functools.partial(
    pl.kernel,
    out_type=jax.ShapeDtypeStruct((2, NPAD, D), jnp.float32),
    mesh=_SC_MESH,
    scratch_types=[
        pltpu.VMEM((GC, CHUNK), jnp.int32),
        pltpu.VMEM((GC, CHUNK), jnp.int32),
        pltpu.VMEM((GC, CHUNK), jnp.int32),
        pltpu.VMEM((GC, CHUNK), jnp.int32),
        pltpu.VMEM((CHUNK, D), jnp.float32),
        pltpu.VMEM((CHUNK, D), jnp.float32),
        pltpu.VMEM_SHARED((NPAD, D), jnp.float32),
        pltpu.SemaphoreType.DMA,
        pltpu.SemaphoreType.DMA,
        pltpu.SemaphoreType.DMA,
        pltpu.SemaphoreType.DMA,
        pltpu.SemaphoreType.DMA,
        pltpu.SemaphoreType.DMA,
    ],
)(_agg_body)


def _scale_body(x_ref, deg_ref, wz_ref, lzt_ref, bz_ref, lzb_ref,
                wh_ref, lht_ref, bh_ref, lhb_ref,
                y_ref, wzp_ref, bzp_ref, whp_ref, bhp_ref):
  deg = deg_ref[0] + deg_ref[1]
  dinv = jnp.where(deg > 0.0, lax.rsqrt(deg), 0.0)
  d = jnp.reshape(dinv, (NPAD, 1))[:N]
  y_ref[...] = x_ref[...] * d
  lzt = lzt_ref[...]
  lht = lht_ref[...]
  wzp_ref[...] = jnp.dot(wz_ref[...], lzt, preferred_element_type=jnp.float32)
  bzp_ref[...] = jnp.dot(bz_ref[...], lzt,
                         preferred_element_type=jnp.float32) + lzb_ref[...]
  whp_ref[...] = jnp.dot(wh_ref[...], lht, preferred_element_type=jnp.float32)
  bhp_ref[...] = jnp.dot(bh_ref[...], lht,
                         preferred_element_type=jnp.float32) + lhb_ref[...]


_scale_call = pl.pallas_call(
    _scale_body,
    out_shape=(
        jax.ShapeDtypeStruct((N, D), jnp.float32),
        jax.ShapeDtypeStruct((D, D), jnp.float32),
        jax.ShapeDtypeStruct((1, D), jnp.float32),
        jax.ShapeDtypeStruct((D, D), jnp.float32),
        jax.ShapeDtypeStruct((1, D), jnp.float32),
    ),
)


_TBLK = 2048


def _tail_body(acc_ref, deg_ref, wzp_ref, bzp_ref, whp_ref, bhp_ref,
               wo_ref, bo_ref, out_ref):
  i = pl.program_id(0)
  deg = (deg_ref[0, pl.ds(i * _TBLK, _TBLK)]
         + deg_ref[1, pl.ds(i * _TBLK, _TBLK)])
  dinv = jnp.where(deg > 0.0, lax.rsqrt(deg), 0.0)
  d = jnp.reshape(dinv, (_TBLK, 1))
  p = (acc_ref[0] + acc_ref[1]) * d
  z = jax.nn.sigmoid(jnp.dot(p, wzp_ref[...],
                             preferred_element_type=jnp.float32) + bzp_ref[...])
  ht = jnp.tanh(jnp.dot(p, whp_ref[...],
                        preferred_element_type=jnp.float32) + bhp_ref[...])
  g = jnp.maximum((1.0 - z) * ht, 0.0)
  out_ref[...] = jnp.dot(g, wo_ref[...],
                         preferred_element_type=jnp.float32) + bo_ref[...]


_tail_call = pl.pallas_call(
    _tail_body,
    grid=(NPAD // _TBLK,),
    in_specs=[
        pl.BlockSpec((2, _TBLK, D), lambda i: (0, i, 0)),
        pl.BlockSpec((2, NPAD), lambda i: (0, 0)),
        pl.BlockSpec((D, D), lambda i: (0, 0)),
        pl.BlockSpec((1, D), lambda i: (0, 0)),
        pl.BlockSpec((D, D), lambda i: (0, 0)),
        pl.BlockSpec((1, D), lambda i: (0, 0)),
        pl.BlockSpec((D, D), lambda i: (0, 0)),
        pl.BlockSpec((1, D), lambda i: (0, 0)),
    ],
    out_specs=pl.BlockSpec((_TBLK, D), lambda i: (i, 0)),
    out_shape=jax.ShapeDtypeStruct((N, D), jnp.float32),
)


def kernel(x, edge_index, Wz, bz, Wr, br, Wh, bh, Lz_w, Lz_b, Lr_w, Lr_b,
           Lh_w, Lh_b, W_out, b_out):
  ei3 = edge_index.reshape(2, NROWS, CHUNK)
  degp = _deg_call(ei3)
  y, wzp, bzp, whp, bhp = _scale_call(
      x, degp, Wz, Lz_w[:D], bz.reshape(1, D),
      Lz_b.reshape(1, D), Wh, Lh_w[:D], bh.reshape(1, D), Lh_b.reshape(1, D))
  accp = _agg_call(y, ei3)
  return _tail_call(accp, degp, wzp, bzp, whp, bhp, W_out, b_out.reshape(1, D))


# speedup vs baseline: 1.1408x; 1.1408x over previous
import functools

import jax
import jax.numpy as jnp
from jax import lax
from jax.experimental import pallas as pl
from jax.experimental.pallas import tpu as pltpu
from jax.experimental.pallas import tpu_sc as plsc

N = 10000
NPAD = 10240
E = 320000
D = 128
CHUNK = 128
NROWS = E // CHUNK
WCH = 80
LASTW = 31
LCH = NROWS - LASTW * WCH
GC = 16
NG = WCH // GC
SLAB = NPAD // 16
_SC_MESH = plsc.VectorSubcoreMesh(core_axis_name="c", subcore_axis_name="s")


def _deg_body(dst3, out, dst_v, ones_v, zb_v, deg_sh, sem):
  c = lax.axis_index("c")
  s = lax.axis_index("s")
  w = c * 16 + s

  @pl.when(w < LASTW)
  def _():
    pltpu.sync_copy(dst3.at[pl.ds(w * WCH, WCH)], dst_v)

  @pl.when(w == LASTW)
  def _():
    pltpu.sync_copy(dst3.at[pl.ds(LASTW * WCH, GC)], dst_v.at[pl.ds(0, GC)])
    pltpu.sync_copy(dst3.at[pl.ds(LASTW * WCH + GC, LCH - GC)],
                    dst_v.at[pl.ds(GC, LCH - GC)])

  for t in range(8):
    ones_v[pl.ds(t * 16, 16)] = jnp.ones((16,), jnp.float32)

  @pl.loop(0, SLAB // 16)
  def _zb(i):
    zb_v[pl.ds(i * 16, 16)] = jnp.zeros((16,), jnp.float32)

  pltpu.sync_copy(zb_v, deg_sh.at[pl.ds(s * SLAB, SLAB)])
  plsc.subcore_barrier()

  nch = jnp.where(w == LASTW, LCH, WCH)

  @pl.loop(0, nch)
  def _fire(j):
    pltpu.async_copy(ones_v, deg_sh.at[dst_v.at[j]], sem, add=True)

  @pl.loop(0, nch)
  def _drain(j):
    pltpu.make_async_copy(ones_v, deg_sh.at[dst_v.at[0]], sem).wait()

  plsc.subcore_barrier()
  pltpu.sync_copy(deg_sh.at[pl.ds(s * SLAB, SLAB)],
                  out.at[c, pl.ds(s * SLAB, SLAB)])


_deg_call = functools.partial(
    pl.kernel,
    out_type=jax.ShapeDtypeStruct((2, NPAD), jnp.float32),
    mesh=_SC_MESH,
    scratch_types=[
        pltpu.VMEM((WCH, CHUNK), jnp.int32),
        pltpu.VMEM((CHUNK,), jnp.float32),
        pltpu.VMEM((SLAB,), jnp.float32),
        pltpu.VMEM_SHARED((NPAD,), jnp.float32),
        pltpu.SemaphoreType.DMA,
    ],
)(_deg_body)


def _agg_body(y, src3, dst3, out, src_a, dst_a, src_b, dst_b, buf0, buf1,
              acc_sh, sg0, sg1, ss0, ss1, sia, sib):
  c = lax.axis_index("c")
  s = lax.axis_index("s")
  w = c * 16 + s

  idx = [(src_a, dst_a, sia), (src_b, dst_b, sib)]

  def refill(g):
    sv, dv, sem = idx[g % 2]
    pltpu.async_copy(src3.at[pl.ds(w * WCH + g * GC, GC)], sv, sem)
    pltpu.async_copy(dst3.at[pl.ds(w * WCH + g * GC, GC)], dv, sem)

  def refill_wait(g):
    sv, dv, sem = idx[g % 2]
    pltpu.make_async_copy(src3.at[pl.ds(0, GC)], sv, sem).wait()
    pltpu.make_async_copy(dst3.at[pl.ds(0, GC)], dv, sem).wait()

  @pl.when(w < LASTW)
  def _prefetch():
    refill(0)
    refill(1)

  @pl.loop(0, CHUNK)
  def _zrow(r):
    for t in range(8):
      buf0[r, pl.ds(t * 16, 16)] = jnp.zeros((16,), jnp.float32)

  for k in range(SLAB // CHUNK):
    pltpu.async_copy(buf0, acc_sh.at[pl.ds(s * SLAB + k * CHUNK, CHUNK)], ss0)
  for k in range(SLAB // CHUNK):
    pltpu.make_async_copy(buf0, acc_sh.at[pl.ds(0, CHUNK)], ss0).wait()
  plsc.subcore_barrier()

  @pl.when(w < LASTW)
  def _full_worker():
    refill_wait(0)
    pltpu.async_copy(y.at[src_a.at[0]], buf0, sg0)
    pltpu.async_copy(y.at[src_a.at[1]], buf1, sg1)

    for g in range(NG):
      sv, dv, _ = idx[g % 2]
      nsv, ndv, _ = idx[(g + 1) % 2]

      @pl.loop(0, GC // 2 - 1)
      def _pipe(i, sv=sv, dv=dv):
        a = 2 * i
        b = a + 1
        pltpu.make_async_copy(y.at[sv.at[0]], buf0, sg0).wait()
        pltpu.async_copy(buf0, acc_sh.at[dv.at[a]], ss0, add=True)
        pltpu.make_async_copy(y.at[sv.at[1]], buf1, sg1).wait()
        pltpu.make_async_copy(buf0, acc_sh.at[dv.at[0]], ss0).wait()
        pltpu.async_copy(y.at[sv.at[a + 2]], buf0, sg0)
        pltpu.async_copy(buf1, acc_sh.at[dv.at[b]], ss1, add=True)
        pltpu.make_async_copy(buf1, acc_sh.at[dv.at[0]], ss1).wait()
        pltpu.async_copy(y.at[sv.at[b + 2]], buf1, sg1)

      pltpu.make_async_copy(y.at[sv.at[0]], buf0, sg0).wait()
      pltpu.async_copy(buf0, acc_sh.at[dv.at[GC - 2]], ss0, add=True)
      pltpu.make_async_copy(y.at[sv.at[1]], buf1, sg1).wait()
      pltpu.make_async_copy(buf0, acc_sh.at[dv.at[0]], ss0).wait()
      if g + 1 < NG:
        refill_wait(g + 1)
        pltpu.async_copy(y.at[nsv.at[0]], buf0, sg0)
      pltpu.async_copy(buf1, acc_sh.at[dv.at[GC - 1]], ss1, add=True)
      pltpu.make_async_copy(buf1, acc_sh.at[dv.at[0]], ss1).wait()
      if g + 1 < NG:
        pltpu.async_copy(y.at[nsv.at[1]], buf1, sg1)
        if g + 2 < NG:
          refill(g + 2)

  @pl.when(w == LASTW)
  def _tail_worker():
    base = LASTW * WCH
    pltpu.sync_copy(src3.at[pl.ds(base, GC)], src_a)
    pltpu.sync_copy(dst3.at[pl.ds(base, GC)], dst_a)
    pltpu.sync_copy(src3.at[pl.ds(base + GC, LCH - GC)],
                    src_b.at[pl.ds(0, LCH - GC)])
    pltpu.sync_copy(dst3.at[pl.ds(base + GC, LCH - GC)],
                    dst_b.at[pl.ds(0, LCH - GC)])

    @pl.loop(0, GC)
    def _t0(j):
      pltpu.async_copy(y.at[src_a.at[j]], buf0, sg0).wait()
      pltpu.sync_copy(buf0, acc_sh.at[dst_a.at[j]], add=True)

    @pl.loop(0, LCH - GC)
    def _t1(j):
      pltpu.async_copy(y.at[src_b.at[j]], buf0, sg0).wait()
      pltpu.sync_copy(buf0, acc_sh.at[dst_b.at[j]], add=True)

  plsc.subcore_barrier()
  pltpu.sync_copy(acc_sh.at[pl.ds(s * SLAB, SLAB)],
                  out.at[c, pl.ds(s * SLAB, SLAB)])


_agg_call = functools.partial(
    pl.kernel,
    out_type=jax.ShapeDtypeStruct((2, NPAD, D), jnp.float32),
    mesh=_SC_MESH,
    scratch_types=[
        pltpu.VMEM((GC, CHUNK), jnp.int32),
        pltpu.VMEM((GC, CHUNK), jnp.int32),
        pltpu.VMEM((GC, CHUNK), jnp.int32),
        pltpu.VMEM((GC, CHUNK), jnp.int32),
        pltpu.VMEM((CHUNK, D), jnp.float32),
        pltpu.VMEM((CHUNK, D), jnp.float32),
        pltpu.VMEM_SHARED((NPAD, D), jnp.float32),
        pltpu.SemaphoreType.DMA,
        pltpu.SemaphoreType.DMA,
        pltpu.SemaphoreType.DMA,
        pltpu.SemaphoreType.DMA,
        pltpu.SemaphoreType.DMA,
        pltpu.SemaphoreType.DMA,
    ],
)(_agg_body)


def _scale_body(x_ref, deg_ref, wz_ref, lzt_ref, bz_ref, lzb_ref,
                wh_ref, lht_ref, bh_ref, lhb_ref,
                y_ref, wzp_ref, bzp_ref, whp_ref, bhp_ref):
  deg = deg_ref[0] + deg_ref[1]
  dinv = jnp.where(deg > 0.0, lax.rsqrt(deg), 0.0)
  d = jnp.reshape(dinv, (NPAD, 1))[:N]
  y_ref[...] = x_ref[...] * d
  lzt = lzt_ref[...]
  lht = lht_ref[...]
  wzp_ref[...] = jnp.dot(wz_ref[...], lzt, preferred_element_type=jnp.float32)
  bzp_ref[...] = jnp.dot(bz_ref[...], lzt,
                         preferred_element_type=jnp.float32) + lzb_ref[...]
  whp_ref[...] = jnp.dot(wh_ref[...], lht, preferred_element_type=jnp.float32)
  bhp_ref[...] = jnp.dot(bh_ref[...], lht,
                         preferred_element_type=jnp.float32) + lhb_ref[...]


_scale_call = pl.pallas_call(
    _scale_body,
    out_shape=(
        jax.ShapeDtypeStruct((N, D), jnp.float32),
        jax.ShapeDtypeStruct((D, D), jnp.float32),
        jax.ShapeDtypeStruct((1, D), jnp.float32),
        jax.ShapeDtypeStruct((D, D), jnp.float32),
        jax.ShapeDtypeStruct((1, D), jnp.float32),
    ),
)


_TBLK = 2048


def _tail_body(acc_ref, deg_ref, wzp_ref, bzp_ref, whp_ref, bhp_ref,
               wo_ref, bo_ref, out_ref):
  i = pl.program_id(0)
  deg = (deg_ref[0, pl.ds(i * _TBLK, _TBLK)]
         + deg_ref[1, pl.ds(i * _TBLK, _TBLK)])
  dinv = jnp.where(deg > 0.0, lax.rsqrt(deg), 0.0)
  d = jnp.reshape(dinv, (_TBLK, 1))
  p = (acc_ref[0] + acc_ref[1]) * d
  z = jax.nn.sigmoid(jnp.dot(p, wzp_ref[...],
                             preferred_element_type=jnp.float32) + bzp_ref[...])
  ht = jnp.tanh(jnp.dot(p, whp_ref[...],
                        preferred_element_type=jnp.float32) + bhp_ref[...])
  g = jnp.maximum((1.0 - z) * ht, 0.0)
  out_ref[...] = jnp.dot(g, wo_ref[...],
                         preferred_element_type=jnp.float32) + bo_ref[...]


_tail_call = pl.pallas_call(
    _tail_body,
    grid=(NPAD // _TBLK,),
    in_specs=[
        pl.BlockSpec((2, _TBLK, D), lambda i: (0, i, 0)),
        pl.BlockSpec((2, NPAD), lambda i: (0, 0)),
        pl.BlockSpec((D, D), lambda i: (0, 0)),
        pl.BlockSpec((1, D), lambda i: (0, 0)),
        pl.BlockSpec((D, D), lambda i: (0, 0)),
        pl.BlockSpec((1, D), lambda i: (0, 0)),
        pl.BlockSpec((D, D), lambda i: (0, 0)),
        pl.BlockSpec((1, D), lambda i: (0, 0)),
    ],
    out_specs=pl.BlockSpec((_TBLK, D), lambda i: (i, 0)),
    out_shape=jax.ShapeDtypeStruct((N, D), jnp.float32),
)


def kernel(x, edge_index, Wz, bz, Wr, br, Wh, bh, Lz_w, Lz_b, Lr_w, Lr_b,
           Lh_w, Lh_b, W_out, b_out):
  dst3 = edge_index[1].reshape(NROWS, CHUNK)
  src3 = edge_index[0].reshape(NROWS, CHUNK)
  degp = _deg_call(dst3)
  y, wzp, bzp, whp, bhp = _scale_call(
      x, degp, Wz, Lz_w[:D], bz.reshape(1, D),
      Lz_b.reshape(1, D), Wh, Lh_w[:D], bh.reshape(1, D), Lh_b.reshape(1, D))
  accp = _agg_call(y, src3, dst3)
  return _tail_call(accp, degp, wzp, bzp, whp, bhp, W_out, b_out.reshape(1, D))


# speedup vs baseline: 1.2262x; 1.0749x over previous
import functools

import jax
import jax.numpy as jnp
from jax import lax
from jax.experimental import pallas as pl
from jax.experimental.pallas import tpu as pltpu
from jax.experimental.pallas import tpu_sc as plsc

N = 10000
NPAD = 10240
E = 320000
D = 128
CHUNK = 128
NROWS = E // CHUNK
WCH = 80
LASTW = 31
LCH = NROWS - LASTW * WCH
GC = 16
NG = WCH // GC
SLAB = NPAD // 16
_SC_MESH = plsc.VectorSubcoreMesh(core_axis_name="c", subcore_axis_name="s")


def _deg_body(ei3, out, dst_v, ones_v, zb_v, deg_sh, sem):
  c = lax.axis_index("c")
  s = lax.axis_index("s")
  w = c * 16 + s

  @pl.when(w < LASTW)
  def _():
    pltpu.sync_copy(ei3.at[1, pl.ds(w * WCH, WCH)], dst_v)

  @pl.when(w == LASTW)
  def _():
    pltpu.sync_copy(ei3.at[1, pl.ds(LASTW * WCH, GC)], dst_v.at[pl.ds(0, GC)])
    pltpu.sync_copy(ei3.at[1, pl.ds(LASTW * WCH + GC, LCH - GC)],
                    dst_v.at[pl.ds(GC, LCH - GC)])

  for t in range(8):
    ones_v[pl.ds(t * 16, 16)] = jnp.ones((16,), jnp.float32)

  @pl.loop(0, SLAB // 16)
  def _zb(i):
    zb_v[pl.ds(i * 16, 16)] = jnp.zeros((16,), jnp.float32)

  pltpu.sync_copy(zb_v, deg_sh.at[pl.ds(s * SLAB, SLAB)])
  plsc.subcore_barrier()

  nch = jnp.where(w == LASTW, LCH, WCH)

  @pl.loop(0, nch)
  def _fire(j):
    pltpu.async_copy(ones_v, deg_sh.at[dst_v.at[j]], sem, add=True)

  @pl.loop(0, nch)
  def _drain(j):
    pltpu.make_async_copy(ones_v, deg_sh.at[dst_v.at[0]], sem).wait()

  plsc.subcore_barrier()
  pltpu.sync_copy(deg_sh.at[pl.ds(s * SLAB, SLAB)],
                  out.at[c, pl.ds(s * SLAB, SLAB)])


_deg_call = functools.partial(
    pl.kernel,
    out_type=jax.ShapeDtypeStruct((2, NPAD), jnp.float32),
    mesh=_SC_MESH,
    scratch_types=[
        pltpu.VMEM((WCH, CHUNK), jnp.int32),
        pltpu.VMEM((CHUNK,), jnp.float32),
        pltpu.VMEM((SLAB,), jnp.float32),
        pltpu.VMEM_SHARED((NPAD,), jnp.float32),
        pltpu.SemaphoreType.DMA,
    ],
)(_deg_body)


def _agg_body(y, ei3, out, src_a, dst_a, src_b, dst_b, buf0, buf1, acc_sh,
              sg0, sg1, ss0, ss1, sia, sib):
  c = lax.axis_index("c")
  s = lax.axis_index("s")
  w = c * 16 + s

  idx = [(src_a, dst_a, sia), (src_b, dst_b, sib)]

  def refill(g):
    sv, dv, sem = idx[g % 2]
    pltpu.async_copy(ei3.at[0, pl.ds(w * WCH + g * GC, GC)], sv, sem)
    pltpu.async_copy(ei3.at[1, pl.ds(w * WCH + g * GC, GC)], dv, sem)

  def refill_wait(g):
    sv, dv, sem = idx[g % 2]
    pltpu.make_async_copy(ei3.at[0, pl.ds(0, GC)], sv, sem).wait()
    pltpu.make_async_copy(ei3.at[1, pl.ds(0, GC)], dv, sem).wait()

  @pl.when(w < LASTW)
  def _prefetch():
    refill(0)
    refill(1)

  @pl.loop(0, CHUNK)
  def _zrow(r):
    for t in range(8):
      buf0[r, pl.ds(t * 16, 16)] = jnp.zeros((16,), jnp.float32)

  for k in range(SLAB // CHUNK):
    pltpu.sync_copy(buf0, acc_sh.at[pl.ds(s * SLAB + k * CHUNK, CHUNK)])
  plsc.subcore_barrier()

  @pl.when(w < LASTW)
  def _full_worker():
    refill_wait(0)
    pltpu.async_copy(y.at[src_a.at[0]], buf0, sg0)
    pltpu.async_copy(y.at[src_a.at[1]], buf1, sg1)

    for g in range(NG):
      sv, dv, _ = idx[g % 2]
      nsv, ndv, _ = idx[(g + 1) % 2]

      @pl.loop(0, GC // 2 - 1)
      def _pipe(i, sv=sv, dv=dv):
        a = 2 * i
        b = a + 1
        pltpu.make_async_copy(y.at[sv.at[0]], buf0, sg0).wait()
        pltpu.async_copy(buf0, acc_sh.at[dv.at[a]], ss0, add=True)
        pltpu.make_async_copy(y.at[sv.at[1]], buf1, sg1).wait()
        pltpu.make_async_copy(buf0, acc_sh.at[dv.at[0]], ss0).wait()
        pltpu.async_copy(y.at[sv.at[a + 2]], buf0, sg0)
        pltpu.async_copy(buf1, acc_sh.at[dv.at[b]], ss1, add=True)
        pltpu.make_async_copy(buf1, acc_sh.at[dv.at[0]], ss1).wait()
        pltpu.async_copy(y.at[sv.at[b + 2]], buf1, sg1)

      pltpu.make_async_copy(y.at[sv.at[0]], buf0, sg0).wait()
      pltpu.async_copy(buf0, acc_sh.at[dv.at[GC - 2]], ss0, add=True)
      pltpu.make_async_copy(y.at[sv.at[1]], buf1, sg1).wait()
      pltpu.make_async_copy(buf0, acc_sh.at[dv.at[0]], ss0).wait()
      if g + 1 < NG:
        refill_wait(g + 1)
        pltpu.async_copy(y.at[nsv.at[0]], buf0, sg0)
      pltpu.async_copy(buf1, acc_sh.at[dv.at[GC - 1]], ss1, add=True)
      pltpu.make_async_copy(buf1, acc_sh.at[dv.at[0]], ss1).wait()
      if g + 1 < NG:
        pltpu.async_copy(y.at[nsv.at[1]], buf1, sg1)
        if g + 2 < NG:
          refill(g + 2)

  @pl.when(w == LASTW)
  def _tail_worker():
    base = LASTW * WCH
    pltpu.sync_copy(ei3.at[0, pl.ds(base, GC)], src_a)
    pltpu.sync_copy(ei3.at[1, pl.ds(base, GC)], dst_a)
    pltpu.sync_copy(ei3.at[0, pl.ds(base + GC, LCH - GC)],
                    src_b.at[pl.ds(0, LCH - GC)])
    pltpu.sync_copy(ei3.at[1, pl.ds(base + GC, LCH - GC)],
                    dst_b.at[pl.ds(0, LCH - GC)])

    @pl.loop(0, GC)
    def _t0(j):
      pltpu.async_copy(y.at[src_a.at[j]], buf0, sg0).wait()
      pltpu.sync_copy(buf0, acc_sh.at[dst_a.at[j]], add=True)

    @pl.loop(0, LCH - GC)
    def _t1(j):
      pltpu.async_copy(y.at[src_b.at[j]], buf0, sg0).wait()
      pltpu.sync_copy(buf0, acc_sh.at[dst_b.at[j]], add=True)

  plsc.subcore_barrier()
  pltpu.sync_copy(acc_sh.at[pl.ds(s * SLAB, SLAB)],
                  out.at[c, pl.ds(s * SLAB, SLAB)])


_agg_call = functools.partial(
    pl.kernel,
    out_type=jax.ShapeDtypeStruct((2, NPAD, D), jnp.float32),
    mesh=_SC_MESH,
    scratch_types=[
        pltpu.VMEM((GC, CHUNK), jnp.int32),
        pltpu.VMEM((GC, CHUNK), jnp.int32),
        pltpu.VMEM((GC, CHUNK), jnp.int32),
        pltpu.VMEM((GC, CHUNK), jnp.int32),
        pltpu.VMEM((CHUNK, D), jnp.float32),
        pltpu.VMEM((CHUNK, D), jnp.float32),
        pltpu.VMEM_SHARED((NPAD, D), jnp.float32),
        pltpu.SemaphoreType.DMA,
        pltpu.SemaphoreType.DMA,
        pltpu.SemaphoreType.DMA,
        pltpu.SemaphoreType.DMA,
        pltpu.SemaphoreType.DMA,
        pltpu.SemaphoreType.DMA,
    ],
)(_agg_body)


def _scale_body(x_ref, deg_ref, wz_ref, lzt_ref, bz_ref, lzb_ref,
                wh_ref, lht_ref, bh_ref, lhb_ref,
                y_ref, wzp_ref, bzp_ref, whp_ref, bhp_ref):
  deg = deg_ref[0] + deg_ref[1]
  dinv = jnp.where(deg > 0.0, lax.rsqrt(deg), 0.0)
  d = jnp.reshape(dinv, (NPAD, 1))[:N]
  y_ref[...] = x_ref[...] * d
  lzt = lzt_ref[...]
  lht = lht_ref[...]
  wzp_ref[...] = jnp.dot(wz_ref[...], lzt, preferred_element_type=jnp.float32)
  bzp_ref[...] = jnp.dot(bz_ref[...], lzt,
                         preferred_element_type=jnp.float32) + lzb_ref[...]
  whp_ref[...] = jnp.dot(wh_ref[...], lht, preferred_element_type=jnp.float32)
  bhp_ref[...] = jnp.dot(bh_ref[...], lht,
                         preferred_element_type=jnp.float32) + lhb_ref[...]


_scale_call = pl.pallas_call(
    _scale_body,
    out_shape=(
        jax.ShapeDtypeStruct((N, D), jnp.float32),
        jax.ShapeDtypeStruct((D, D), jnp.float32),
        jax.ShapeDtypeStruct((1, D), jnp.float32),
        jax.ShapeDtypeStruct((D, D), jnp.float32),
        jax.ShapeDtypeStruct((1, D), jnp.float32),
    ),
)


_TBLK = 2048


def _tail_body(acc_ref, deg_ref, wzp_ref, bzp_ref, whp_ref, bhp_ref,
               wo_ref, bo_ref, out_ref):
  i = pl.program_id(0)
  deg = (deg_ref[0, pl.ds(i * _TBLK, _TBLK)]
         + deg_ref[1, pl.ds(i * _TBLK, _TBLK)])
  dinv = jnp.where(deg > 0.0, lax.rsqrt(deg), 0.0)
  d = jnp.reshape(dinv, (_TBLK, 1))
  p = (acc_ref[0] + acc_ref[1]) * d
  z = jax.nn.sigmoid(jnp.dot(p, wzp_ref[...],
                             preferred_element_type=jnp.float32) + bzp_ref[...])
  ht = jnp.tanh(jnp.dot(p, whp_ref[...],
                        preferred_element_type=jnp.float32) + bhp_ref[...])
  g = jnp.maximum((1.0 - z) * ht, 0.0)
  out_ref[...] = jnp.dot(g, wo_ref[...],
                         preferred_element_type=jnp.float32) + bo_ref[...]


_tail_call = pl.pallas_call(
    _tail_body,
    grid=(NPAD // _TBLK,),
    in_specs=[
        pl.BlockSpec((2, _TBLK, D), lambda i: (0, i, 0)),
        pl.BlockSpec((2, NPAD), lambda i: (0, 0)),
        pl.BlockSpec((D, D), lambda i: (0, 0)),
        pl.BlockSpec((1, D), lambda i: (0, 0)),
        pl.BlockSpec((D, D), lambda i: (0, 0)),
        pl.BlockSpec((1, D), lambda i: (0, 0)),
        pl.BlockSpec((D, D), lambda i: (0, 0)),
        pl.BlockSpec((1, D), lambda i: (0, 0)),
    ],
    out_specs=pl.BlockSpec((_TBLK, D), lambda i: (i, 0)),
    out_shape=jax.ShapeDtypeStruct((N, D), jnp.float32),
)


def kernel(x, edge_index, Wz, bz, Wr, br, Wh, bh, Lz_w, Lz_b, Lr_w, Lr_b,
           Lh_w, Lh_b, W_out, b_out):
  ei3 = edge_index.reshape(2, NROWS, CHUNK)
  degp = _deg_call(ei3)
  y, wzp, bzp, whp, bhp = _scale_call(
      x, degp, Wz, Lz_w[:D], bz.reshape(1, D),
      Lz_b.reshape(1, D), Wh, Lh_w[:D], bh.reshape(1, D), Lh_b.reshape(1, D))
  accp = _agg_call(y, ei3)
  return _tail_call(accp, degp, wzp, bzp, whp, bhp, W_out, b_out.reshape(1, D))


# speedup vs baseline: 1.2267x; 1.0004x over previous
import functools

import jax
import jax.numpy as jnp
from jax import lax
from jax.experimental import pallas as pl
from jax.experimental.pallas import tpu as pltpu
from jax.experimental.pallas import tpu_sc as plsc

N = 10000
NPAD = 10240
E = 320000
D = 128
CHUNK = 128
NROWS = E // CHUNK
WCH = 80
LASTW = 31
LCH = NROWS - LASTW * WCH
GC = 16
NG = WCH // GC
SLAB = NPAD // 16
_SC_MESH = plsc.VectorSubcoreMesh(core_axis_name="c", subcore_axis_name="s")


def _deg_body(ei3, out, dst_v, ones_v, zb_v, deg_sh, sem):
  c = lax.axis_index("c")
  s = lax.axis_index("s")
  w = c * 16 + s

  @pl.when(w < LASTW)
  def _():
    pltpu.async_copy(ei3.at[1, pl.ds(w * WCH, WCH)], dst_v, sem)

  @pl.when(w == LASTW)
  def _():
    pltpu.async_copy(ei3.at[1, pl.ds(LASTW * WCH, GC)],
                     dst_v.at[pl.ds(0, GC)], sem)
    pltpu.async_copy(ei3.at[1, pl.ds(LASTW * WCH + GC, LCH - GC)],
                     dst_v.at[pl.ds(GC, LCH - GC)], sem)

  for t in range(8):
    ones_v[pl.ds(t * 16, 16)] = jnp.ones((16,), jnp.float32)

  @pl.loop(0, SLAB // 16)
  def _zb(i):
    zb_v[pl.ds(i * 16, 16)] = jnp.zeros((16,), jnp.float32)

  pltpu.sync_copy(zb_v, deg_sh.at[pl.ds(s * SLAB, SLAB)])

  @pl.when(w < LASTW)
  def _():
    pltpu.make_async_copy(ei3.at[1, pl.ds(0, WCH)], dst_v, sem).wait()

  @pl.when(w == LASTW)
  def _():
    pltpu.make_async_copy(ei3.at[1, pl.ds(0, GC)],
                          dst_v.at[pl.ds(0, GC)], sem).wait()
    pltpu.make_async_copy(ei3.at[1, pl.ds(0, LCH - GC)],
                          dst_v.at[pl.ds(GC, LCH - GC)], sem).wait()

  plsc.subcore_barrier()

  nch = jnp.where(w == LASTW, LCH, WCH)

  @pl.loop(0, nch)
  def _fire(j):
    pltpu.async_copy(ones_v, deg_sh.at[dst_v.at[j]], sem, add=True)

  @pl.loop(0, nch)
  def _drain(j):
    pltpu.make_async_copy(ones_v, deg_sh.at[dst_v.at[0]], sem).wait()

  plsc.subcore_barrier()
  pltpu.sync_copy(deg_sh.at[pl.ds(s * SLAB, SLAB)],
                  out.at[c, pl.ds(s * SLAB, SLAB)])


_deg_call = functools.partial(
    pl.kernel,
    out_type=jax.ShapeDtypeStruct((2, NPAD), jnp.float32),
    mesh=_SC_MESH,
    scratch_types=[
        pltpu.VMEM((WCH, CHUNK), jnp.int32),
        pltpu.VMEM((CHUNK,), jnp.float32),
        pltpu.VMEM((SLAB,), jnp.float32),
        pltpu.VMEM_SHARED((NPAD,), jnp.float32),
        pltpu.SemaphoreType.DMA,
    ],
)(_deg_body)


def _agg_body(y, ei3, out, src_a, dst_a, src_b, dst_b, buf0, buf1, acc_sh,
              sg0, sg1, ss0, ss1, sia, sib):
  c = lax.axis_index("c")
  s = lax.axis_index("s")
  w = c * 16 + s

  idx = [(src_a, dst_a, sia), (src_b, dst_b, sib)]

  def refill(g):
    sv, dv, sem = idx[g % 2]
    pltpu.async_copy(ei3.at[0, pl.ds(w * WCH + g * GC, GC)], sv, sem)
    pltpu.async_copy(ei3.at[1, pl.ds(w * WCH + g * GC, GC)], dv, sem)

  def refill_wait(g):
    sv, dv, sem = idx[g % 2]
    pltpu.make_async_copy(ei3.at[0, pl.ds(0, GC)], sv, sem).wait()
    pltpu.make_async_copy(ei3.at[1, pl.ds(0, GC)], dv, sem).wait()

  @pl.when(w < LASTW)
  def _prefetch():
    refill(0)
    refill(1)

  @pl.loop(0, CHUNK)
  def _zrow(r):
    for t in range(8):
      buf0[r, pl.ds(t * 16, 16)] = jnp.zeros((16,), jnp.float32)

  for k in range(SLAB // CHUNK):
    pltpu.sync_copy(buf0, acc_sh.at[pl.ds(s * SLAB + k * CHUNK, CHUNK)])
  plsc.subcore_barrier()

  @pl.when(w < LASTW)
  def _full_worker():
    refill_wait(0)
    pltpu.async_copy(y.at[src_a.at[0]], buf0, sg0)
    pltpu.async_copy(y.at[src_a.at[1]], buf1, sg1)

    for g in range(NG):
      sv, dv, _ = idx[g % 2]
      nsv, ndv, _ = idx[(g + 1) % 2]

      @pl.loop(0, GC // 2 - 1)
      def _pipe(i, sv=sv, dv=dv):
        a = 2 * i
        b = a + 1
        pltpu.make_async_copy(y.at[sv.at[0]], buf0, sg0).wait()
        pltpu.async_copy(buf0, acc_sh.at[dv.at[a]], ss0, add=True)
        pltpu.make_async_copy(y.at[sv.at[1]], buf1, sg1).wait()
        pltpu.make_async_copy(buf0, acc_sh.at[dv.at[0]], ss0).wait()
        pltpu.async_copy(y.at[sv.at[a + 2]], buf0, sg0)
        pltpu.async_copy(buf1, acc_sh.at[dv.at[b]], ss1, add=True)
        pltpu.make_async_copy(buf1, acc_sh.at[dv.at[0]], ss1).wait()
        pltpu.async_copy(y.at[sv.at[b + 2]], buf1, sg1)

      pltpu.make_async_copy(y.at[sv.at[0]], buf0, sg0).wait()
      pltpu.async_copy(buf0, acc_sh.at[dv.at[GC - 2]], ss0, add=True)
      pltpu.make_async_copy(y.at[sv.at[1]], buf1, sg1).wait()
      pltpu.make_async_copy(buf0, acc_sh.at[dv.at[0]], ss0).wait()
      if g + 1 < NG:
        refill_wait(g + 1)
        pltpu.async_copy(y.at[nsv.at[0]], buf0, sg0)
      pltpu.async_copy(buf1, acc_sh.at[dv.at[GC - 1]], ss1, add=True)
      pltpu.make_async_copy(buf1, acc_sh.at[dv.at[0]], ss1).wait()
      if g + 1 < NG:
        pltpu.async_copy(y.at[nsv.at[1]], buf1, sg1)
        if g + 2 < NG:
          refill(g + 2)

  @pl.when(w == LASTW)
  def _tail_worker():
    base = LASTW * WCH
    pltpu.sync_copy(ei3.at[0, pl.ds(base, GC)], src_a)
    pltpu.sync_copy(ei3.at[1, pl.ds(base, GC)], dst_a)
    pltpu.sync_copy(ei3.at[0, pl.ds(base + GC, LCH - GC)],
                    src_b.at[pl.ds(0, LCH - GC)])
    pltpu.sync_copy(ei3.at[1, pl.ds(base + GC, LCH - GC)],
                    dst_b.at[pl.ds(0, LCH - GC)])

    @pl.loop(0, GC)
    def _t0(j):
      pltpu.async_copy(y.at[src_a.at[j]], buf0, sg0).wait()
      pltpu.sync_copy(buf0, acc_sh.at[dst_a.at[j]], add=True)

    @pl.loop(0, LCH - GC)
    def _t1(j):
      pltpu.async_copy(y.at[src_b.at[j]], buf0, sg0).wait()
      pltpu.sync_copy(buf0, acc_sh.at[dst_b.at[j]], add=True)

  plsc.subcore_barrier()
  pltpu.sync_copy(acc_sh.at[pl.ds(s * SLAB, SLAB)],
                  out.at[c, pl.ds(s * SLAB, SLAB)])


_agg_call = functools.partial(
    pl.kernel,
    out_type=jax.ShapeDtypeStruct((2, NPAD, D), jnp.float32),
    mesh=_SC_MESH,
    scratch_types=[
        pltpu.VMEM((GC, CHUNK), jnp.int32),
        pltpu.VMEM((GC, CHUNK), jnp.int32),
        pltpu.VMEM((GC, CHUNK), jnp.int32),
        pltpu.VMEM((GC, CHUNK), jnp.int32),
        pltpu.VMEM((CHUNK, D), jnp.float32),
        pltpu.VMEM((CHUNK, D), jnp.float32),
        pltpu.VMEM_SHARED((NPAD, D), jnp.float32),
        pltpu.SemaphoreType.DMA,
        pltpu.SemaphoreType.DMA,
        pltpu.SemaphoreType.DMA,
        pltpu.SemaphoreType.DMA,
        pltpu.SemaphoreType.DMA,
        pltpu.SemaphoreType.DMA,
    ],
)(_agg_body)


def _scale_body(x_ref, deg_ref, wz_ref, lzt_ref, bz_ref, lzb_ref,
                wh_ref, lht_ref, bh_ref, lhb_ref,
                y_ref, wzp_ref, bzp_ref, whp_ref, bhp_ref):
  deg = deg_ref[0] + deg_ref[1]
  dinv = jnp.where(deg > 0.0, lax.rsqrt(deg), 0.0)
  d = jnp.reshape(dinv, (NPAD, 1))[:N]
  y_ref[...] = x_ref[...] * d
  lzt = lzt_ref[...]
  lht = lht_ref[...]
  wzp_ref[...] = jnp.dot(wz_ref[...], lzt, preferred_element_type=jnp.float32)
  bzp_ref[...] = jnp.dot(bz_ref[...], lzt,
                         preferred_element_type=jnp.float32) + lzb_ref[...]
  whp_ref[...] = jnp.dot(wh_ref[...], lht, preferred_element_type=jnp.float32)
  bhp_ref[...] = jnp.dot(bh_ref[...], lht,
                         preferred_element_type=jnp.float32) + lhb_ref[...]


_scale_call = pl.pallas_call(
    _scale_body,
    out_shape=(
        jax.ShapeDtypeStruct((N, D), jnp.float32),
        jax.ShapeDtypeStruct((D, D), jnp.float32),
        jax.ShapeDtypeStruct((1, D), jnp.float32),
        jax.ShapeDtypeStruct((D, D), jnp.float32),
        jax.ShapeDtypeStruct((1, D), jnp.float32),
    ),
)


_TBLK = 2048


def _tail_body(acc_ref, deg_ref, wzp_ref, bzp_ref, whp_ref, bhp_ref,
               wo_ref, bo_ref, out_ref):
  i = pl.program_id(0)
  deg = (deg_ref[0, pl.ds(i * _TBLK, _TBLK)]
         + deg_ref[1, pl.ds(i * _TBLK, _TBLK)])
  dinv = jnp.where(deg > 0.0, lax.rsqrt(deg), 0.0)
  d = jnp.reshape(dinv, (_TBLK, 1))
  p = (acc_ref[0] + acc_ref[1]) * d
  z = jax.nn.sigmoid(jnp.dot(p, wzp_ref[...],
                             preferred_element_type=jnp.float32) + bzp_ref[...])
  ht = jnp.tanh(jnp.dot(p, whp_ref[...],
                        preferred_element_type=jnp.float32) + bhp_ref[...])
  g = jnp.maximum((1.0 - z) * ht, 0.0)
  out_ref[...] = jnp.dot(g, wo_ref[...],
                         preferred_element_type=jnp.float32) + bo_ref[...]


_tail_call = pl.pallas_call(
    _tail_body,
    grid=(NPAD // _TBLK,),
    in_specs=[
        pl.BlockSpec((2, _TBLK, D), lambda i: (0, i, 0)),
        pl.BlockSpec((2, NPAD), lambda i: (0, 0)),
        pl.BlockSpec((D, D), lambda i: (0, 0)),
        pl.BlockSpec((1, D), lambda i: (0, 0)),
        pl.BlockSpec((D, D), lambda i: (0, 0)),
        pl.BlockSpec((1, D), lambda i: (0, 0)),
        pl.BlockSpec((D, D), lambda i: (0, 0)),
        pl.BlockSpec((1, D), lambda i: (0, 0)),
    ],
    out_specs=pl.BlockSpec((_TBLK, D), lambda i: (i, 0)),
    out_shape=jax.ShapeDtypeStruct((N, D), jnp.float32),
)


def kernel(x, edge_index, Wz, bz, Wr, br, Wh, bh, Lz_w, Lz_b, Lr_w, Lr_b,
           Lh_w, Lh_b, W_out, b_out):
  ei3 = edge_index.reshape(2, NROWS, CHUNK)
  degp = _deg_call(ei3)
  y, wzp, bzp, whp, bhp = _scale_call(
      x, degp, Wz, Lz_w[:D], bz.reshape(1, D),
      Lz_b.reshape(1, D), Wh, Lh_w[:D], bh.reshape(1, D), Lh_b.reshape(1, D))
  accp = _agg_call(y, ei3)
  return _tail_call(accp, degp, wzp, bzp, whp, bhp, W_out, b_out.reshape(1, D))
